# Initial kernel scaffold; baseline (speedup 1.0000x reference)
import functools

import jax
import jax.numpy as jnp
from jax import lax
from jax.experimental import pallas as pl
from jax.experimental.pallas import tpu as pltpu
from jax.experimental.pallas import tpu_sc as plsc

_N = 10000
_E = 320000
_D = 128
_R = 8

_NC = 2
_NS = 16
_NW = _NC * _NS

_CH = 128
_CPW = 79
_EPW = _CH * _CPW
_EPAD = _NW * _EPW
_ROWS2D = _EPAD // _CH
_HROWS = _ROWS2D // _NS

_DEGSZ = 98304
_ACCROWS = _N + 16
_TBLROWS = (_R + 1) * _N


def _mm_body(x_ref, w_ref, o_ref):
    o_ref[...] = jnp.dot(x_ref[...], w_ref[0],
                         preferred_element_type=jnp.float32)


def _transform(x, wall):
    nblk = _N // 1000
    return pl.pallas_call(
        _mm_body,
        grid=(_R + 1, nblk),
        in_specs=[
            pl.BlockSpec((1000, _D), lambda r, i: (i, 0)),
            pl.BlockSpec((1, _D, _D), lambda r, i: (r, 0, 0)),
        ],
        out_specs=pl.BlockSpec((1000, _D), lambda r, i: (r * nblk + i, 0)),
        out_shape=jax.ShapeDtypeStruct((_TBLROWS, _D), jnp.float32),
    )(x, wall)


def _final_body(p_ref, t_ref, b_ref, o_ref):
    o_ref[...] = p_ref[0] + p_ref[1] + t_ref[...] + b_ref[...]


def _finalize(parts, table, bias):
    nblk = _N // 1000
    return pl.pallas_call(
        _final_body,
        grid=(nblk,),
        in_specs=[
            pl.BlockSpec((2, 1000, _D), lambda i: (0, i, 0)),
            pl.BlockSpec((1000, _D), lambda i: (_R * nblk + i, 0)),
            pl.BlockSpec((1, _D), lambda i: (0, 0)),
        ],
        out_specs=pl.BlockSpec((1000, _D), lambda i: (i, 0)),
        out_shape=jax.ShapeDtypeStruct((_N, _D), jnp.float32),
    )(parts, table, bias)


def _sc_body(src_hbm, dst_hbm, et_hbm, table_hbm, parts_hbm,
             etbuf, dstbuf, hcomb, normbuf, rowsA, rowsB, ones_v, zline,
             deg_sh, acc_sh):
    c = lax.axis_index("c")
    s = lax.axis_index("s")
    wid = s * _NC + c

    zero16 = jnp.zeros((16,), jnp.float32)

    def _zrow(i, _):
        for k in range(8):
            rowsA[i, pl.ds(k * 16, 16)] = zero16
        return 0
    lax.fori_loop(0, _CH, _zrow, 0)
    for k in range(8):
        ones_v[pl.ds(k * 16, 16)] = jnp.full((16,), 1.0, jnp.float32)
    for k in range(64):
        zline[pl.ds(k * 16, 16)] = zero16

    dslice = _DEGSZ // _NS
    for k in range(6):
        pltpu.sync_copy(zline, deg_sh.at[pl.ds(s * dslice + k * 1024, 1024)])
    aslice = _ACCROWS // _NS
    for k in range(4):
        pltpu.sync_copy(rowsA, acc_sh.at[pl.ds(s * aslice + k * _CH, _CH)])
    pltpu.sync_copy(rowsA.at[pl.ds(0, aslice - 4 * _CH)],
                    acc_sh.at[pl.ds(s * aslice + 4 * _CH, aslice - 4 * _CH)])
    plsc.subcore_barrier()

    h0 = s * _HROWS
    pltpu.sync_copy(et_hbm.at[pl.ds(h0, _HROWS)], etbuf)
    pltpu.sync_copy(dst_hbm.at[pl.ds(h0, _HROWS)], dstbuf)

    def _comb_row(j, _):
        for k in range(8):
            sl = pl.ds(k * 16, 16)
            hcomb[j, sl] = etbuf[j, sl] * _N + dstbuf[j, sl]
        return 0
    lax.fori_loop(0, _HROWS, _comb_row, 0)

    def _hist_row(j, _):
        pltpu.sync_copy(ones_v, deg_sh.at[hcomb.at[j]], add=True)
        return 0
    lax.fori_loop(0, _HROWS, _hist_row, 0)
    plsc.subcore_barrier()

    w0 = wid * _CPW
    pltpu.sync_copy(et_hbm.at[pl.ds(w0, _CPW)], etbuf.at[pl.ds(0, _CPW)])
    pltpu.sync_copy(dst_hbm.at[pl.ds(w0, _CPW)], dstbuf.at[pl.ds(0, _CPW)])
    pltpu.sync_copy(src_hbm.at[pl.ds(w0, _CPW)],
                    dstbuf.at[pl.ds(_CPW, _CPW)])

    def _idx_row(j, _):
        for k in range(8):
            sl = pl.ds(k * 16, 16)
            et = etbuf[j, sl]
            hcomb[j, sl] = et * _N + dstbuf[j, sl]
            etbuf[j, sl] = et * _N + dstbuf[j + _CPW, sl]
        return 0
    lax.fori_loop(0, _CPW, _idx_row, 0)

    def _deg_row(j, _):
        pltpu.sync_copy(deg_sh.at[hcomb.at[j]], normbuf.at[j])
        return 0
    lax.fori_loop(0, _CPW, _deg_row, 0)

    def _norm_row(j, _):
        for k in range(8):
            sl = pl.ds(k * 16, 16)
            normbuf[j, sl] = 1.0 / normbuf[j, sl]
        return 0
    lax.fori_loop(0, _CPW, _norm_row, 0)

    def _edge_chunk(j, _):
        pltpu.sync_copy(table_hbm.at[etbuf.at[j]], rowsA)

        def _scale(e, _):
            n = normbuf[j, e]
            for k in range(8):
                sl = pl.ds(k * 16, 16)
                rowsA[e, sl] = rowsA[e, sl] * n
            return 0
        lax.fori_loop(0, _CH, _scale, 0)
        pltpu.sync_copy(rowsA, acc_sh.at[dstbuf.at[j]], add=True)
        return 0
    lax.fori_loop(0, _CPW, _edge_chunk, 0)
    plsc.subcore_barrier()

    pltpu.sync_copy(acc_sh.at[pl.ds(s * aslice, aslice)],
                    parts_hbm.at[c, pl.ds(s * aslice, aslice)])


_sc_kernel = functools.partial(
    pl.kernel,
    out_type=jax.ShapeDtypeStruct((_NC, _ACCROWS, _D), jnp.float32),
    mesh=plsc.VectorSubcoreMesh(core_axis_name="c", subcore_axis_name="s",
                                num_cores=_NC, num_subcores=_NS),
    scratch_types=[
        pltpu.VMEM((_HROWS, _CH), jnp.int32),
        pltpu.VMEM((2 * _CPW, _CH), jnp.int32),
        pltpu.VMEM((_HROWS, _CH), jnp.int32),
        pltpu.VMEM((_CPW, _CH), jnp.float32),
        pltpu.VMEM((_CH, _D), jnp.float32),
        pltpu.VMEM((_CH, _D), jnp.float32),
        pltpu.VMEM((_CH,), jnp.float32),
        pltpu.VMEM((1024,), jnp.float32),
        pltpu.VMEM_SHARED((_DEGSZ,), jnp.float32),
        pltpu.VMEM_SHARED((_ACCROWS, _D), jnp.float32),
    ],
)(_sc_body)


def kernel(x, edge_index, edge_type, W, root, bias):
    src = edge_index[0]
    dst = edge_index[1]
    npad = _EPAD - _E
    srcp = jnp.pad(src, (0, npad)).reshape(_ROWS2D, _CH)
    dstp = jnp.pad(dst, (0, npad), constant_values=_N).reshape(_ROWS2D, _CH)
    etp = jnp.pad(edge_type, (0, npad),
                  constant_values=_R).reshape(_ROWS2D, _CH)
    wall = jnp.concatenate([W, root[None]], axis=0)

    table = _transform(x, wall)
    parts = _sc_kernel(srcp, dstp, etp, table)
    return _finalize(parts, table, bias.reshape(1, _D))


# speedup vs baseline: 8.4247x; 8.4247x over previous
import functools

import jax
import jax.numpy as jnp
from jax import lax
from jax.experimental import pallas as pl
from jax.experimental.pallas import tpu as pltpu
from jax.experimental.pallas import tpu_sc as plsc

_N = 10000
_E = 320000
_D = 128
_R = 8

_NC = 1
_NS = 16

_CH = 128
_CPS = 160
_EPAD = _NC * _NS * _CPS * _CH
_ROWS2D = _EPAD // _CH

_DEGSZ = 98304
_ACCROWS = 10112
_TBLROWS = (_R + 1) * _N


def _mm_body(x_ref, w_ref, o_ref):
    o_ref[...] = jnp.dot(x_ref[...], w_ref[0],
                         preferred_element_type=jnp.float32)


def _transform(x, wall):
    nblk = _N // 1000
    return pl.pallas_call(
        _mm_body,
        grid=(_R + 1, nblk),
        in_specs=[
            pl.BlockSpec((1000, _D), lambda r, i: (i, 0)),
            pl.BlockSpec((1, _D, _D), lambda r, i: (r, 0, 0)),
        ],
        out_specs=pl.BlockSpec((1000, _D), lambda r, i: (r * nblk + i, 0)),
        out_shape=jax.ShapeDtypeStruct((_TBLROWS, _D), jnp.float32),
    )(x, wall)


def _final_body(p_ref, t_ref, b_ref, o_ref):
    o_ref[...] = p_ref[...] + t_ref[...] + b_ref[...]


def _finalize(parts, table, bias):
    nblk = _N // 1000
    return pl.pallas_call(
        _final_body,
        grid=(nblk,),
        in_specs=[
            pl.BlockSpec((1000, _D), lambda i: (i, 0)),
            pl.BlockSpec((1000, _D), lambda i: (_R * nblk + i, 0)),
            pl.BlockSpec((1, _D), lambda i: (0, 0)),
        ],
        out_specs=pl.BlockSpec((1000, _D), lambda i: (i, 0)),
        out_shape=jax.ShapeDtypeStruct((_N, _D), jnp.float32),
    )(parts, table, bias)


_BR = 32
_NBLK = _CPS // _BR


def _sc_body(src_hbm, dst_hbm, et_hbm, table_hbm, parts_hbm,
             etbuf, dstbuf, auxbuf, normbuf, rows, ones_v, zline,
             deg_sh, acc_sh):
    s = lax.axis_index("s")

    zero16 = jnp.zeros((16,), jnp.float32)

    def _zrow(i, _):
        for k in range(8):
            rows[i, pl.ds(k * 16, 16)] = zero16
        return 0
    lax.fori_loop(0, _CH, _zrow, 0)
    for k in range(8):
        ones_v[pl.ds(k * 16, 16)] = jnp.full((16,), 1.0, jnp.float32)
    for k in range(64):
        zline[pl.ds(k * 16, 16)] = zero16

    dslice = _DEGSZ // _NS
    for k in range(6):
        pltpu.sync_copy(zline, deg_sh.at[pl.ds(s * dslice + k * 1024, 1024)])
    aslice = _ACCROWS // _NS
    for k in range(4):
        pltpu.sync_copy(rows, acc_sh.at[pl.ds(s * aslice + k * _CH, _CH)])
    pltpu.sync_copy(rows.at[pl.ds(0, aslice - 4 * _CH)],
                    acc_sh.at[pl.ds(s * aslice + 4 * _CH, aslice - 4 * _CH)])
    plsc.subcore_barrier()

    h0 = s * _CPS

    def _hist_blk(b, _):
        base = h0 + b * _BR
        pltpu.sync_copy(et_hbm.at[pl.ds(base, _BR)], etbuf)
        pltpu.sync_copy(dst_hbm.at[pl.ds(base, _BR)], dstbuf)

        def _comb_row(j, _):
            for k in range(8):
                sl = pl.ds(k * 16, 16)
                auxbuf[j, sl] = etbuf[j, sl] * _N + dstbuf[j, sl]
            return 0
        lax.fori_loop(0, _BR, _comb_row, 0)

        def _hist_row(j, _):
            pltpu.sync_copy(ones_v, deg_sh.at[auxbuf.at[j]], add=True)
            return 0
        lax.fori_loop(0, _BR, _hist_row, 0)
        return 0
    lax.fori_loop(0, _NBLK, _hist_blk, 0)
    plsc.subcore_barrier()

    def _main_blk(b, _):
        base = h0 + b * _BR
        pltpu.sync_copy(et_hbm.at[pl.ds(base, _BR)], etbuf)
        pltpu.sync_copy(dst_hbm.at[pl.ds(base, _BR)], dstbuf)

        def _comb_row(j, _):
            for k in range(8):
                sl = pl.ds(k * 16, 16)
                auxbuf[j, sl] = etbuf[j, sl] * _N + dstbuf[j, sl]
            return 0
        lax.fori_loop(0, _BR, _comb_row, 0)

        def _deg_row(j, _):
            pltpu.sync_copy(deg_sh.at[auxbuf.at[j]], normbuf.at[j])
            return 0
        lax.fori_loop(0, _BR, _deg_row, 0)

        def _norm_row(j, _):
            for k in range(8):
                sl = pl.ds(k * 16, 16)
                normbuf[j, sl] = 1.0 / normbuf[j, sl]
            return 0
        lax.fori_loop(0, _BR, _norm_row, 0)

        pltpu.sync_copy(src_hbm.at[pl.ds(base, _BR)], auxbuf)

        def _idx_row(j, _):
            for k in range(8):
                sl = pl.ds(k * 16, 16)
                etbuf[j, sl] = etbuf[j, sl] * _N + auxbuf[j, sl]
            return 0
        lax.fori_loop(0, _BR, _idx_row, 0)

        def _edge_chunk(j, _):
            pltpu.sync_copy(table_hbm.at[etbuf.at[j]], rows)

            def _scale(eb, _):
                nv = normbuf[j, pl.ds(eb * 16, 16)]
                for i in range(16):
                    e = eb * 16 + i
                    n = nv[i]
                    for k in range(8):
                        sl = pl.ds(k * 16, 16)
                        rows[e, sl] = rows[e, sl] * n
                return 0
            lax.fori_loop(0, _CH // 16, _scale, 0)
            pltpu.sync_copy(rows, acc_sh.at[dstbuf.at[j]], add=True)
            return 0
        lax.fori_loop(0, _BR, _edge_chunk, 0)
        return 0
    lax.fori_loop(0, _NBLK, _main_blk, 0)
    plsc.subcore_barrier()

    pltpu.sync_copy(acc_sh.at[pl.ds(s * aslice, aslice)],
                    parts_hbm.at[pl.ds(s * aslice, aslice)])


_sc_kernel = functools.partial(
    pl.kernel,
    out_type=jax.ShapeDtypeStruct((_ACCROWS, _D), jnp.float32),
    mesh=plsc.VectorSubcoreMesh(core_axis_name="c", subcore_axis_name="s",
                                num_cores=_NC, num_subcores=_NS),
    scratch_types=[
        pltpu.VMEM((_BR, _CH), jnp.int32),
        pltpu.VMEM((_BR, _CH), jnp.int32),
        pltpu.VMEM((_BR, _CH), jnp.int32),
        pltpu.VMEM((_BR, _CH), jnp.float32),
        pltpu.VMEM((_CH, _D), jnp.float32),
        pltpu.VMEM((_CH,), jnp.float32),
        pltpu.VMEM((1024,), jnp.float32),
        pltpu.VMEM_SHARED((_DEGSZ,), jnp.float32),
        pltpu.VMEM_SHARED((_ACCROWS, _D), jnp.float32),
    ],
)(_sc_body)


def kernel(x, edge_index, edge_type, W, root, bias):
    src = edge_index[0]
    dst = edge_index[1]
    npad = _EPAD - _E
    srcp = jnp.pad(src, (0, npad)).reshape(_ROWS2D, _CH)
    dstp = jnp.pad(dst, (0, npad), constant_values=_N).reshape(_ROWS2D, _CH)
    etp = jnp.pad(edge_type, (0, npad),
                  constant_values=_R).reshape(_ROWS2D, _CH)
    wall = jnp.concatenate([W, root[None]], axis=0)

    table = _transform(x, wall)
    parts = _sc_kernel(srcp, dstp, etp, table)
    return _finalize(parts, table, bias.reshape(1, _D))


# speedup vs baseline: 9.9378x; 1.1796x over previous
import functools

import jax
import jax.numpy as jnp
from jax import lax
from jax.experimental import pallas as pl
from jax.experimental.pallas import tpu as pltpu
from jax.experimental.pallas import tpu_sc as plsc

_N = 10000
_E = 320000
_D = 128
_R = 8

_NC = 1
_NS = 16

_CH = 128
_CPS = 160
_EPAD = _NC * _NS * _CPS * _CH
_ROWS2D = _EPAD // _CH

_DEGSZ = 90112
_ACCROWS = 10112
_TBLROWS = (_R + 1) * _N


def _mm_body(x_ref, w_ref, o_ref):
    o_ref[...] = jnp.dot(x_ref[...], w_ref[0],
                         preferred_element_type=jnp.float32)


def _transform(x, wall):
    nblk = _N // 1000
    return pl.pallas_call(
        _mm_body,
        grid=(_R + 1, nblk),
        in_specs=[
            pl.BlockSpec((1000, _D), lambda r, i: (i, 0)),
            pl.BlockSpec((1, _D, _D), lambda r, i: (r, 0, 0)),
        ],
        out_specs=pl.BlockSpec((1000, _D), lambda r, i: (r * nblk + i, 0)),
        out_shape=jax.ShapeDtypeStruct((_TBLROWS, _D), jnp.float32),
    )(x, wall)


def _final_body(p_ref, t_ref, b_ref, o_ref):
    o_ref[...] = p_ref[...] + t_ref[...] + b_ref[...]


def _finalize(parts, table, bias):
    nblk = _N // 1000
    return pl.pallas_call(
        _final_body,
        grid=(nblk,),
        in_specs=[
            pl.BlockSpec((1000, _D), lambda i: (i, 0)),
            pl.BlockSpec((1000, _D), lambda i: (_R * nblk + i, 0)),
            pl.BlockSpec((1, _D), lambda i: (0, 0)),
        ],
        out_specs=pl.BlockSpec((1000, _D), lambda i: (i, 0)),
        out_shape=jax.ShapeDtypeStruct((_N, _D), jnp.float32),
    )(parts, table, bias)


_BR = 16
_NBLK = _CPS // _BR


def _sc_body(src_hbm, dst_hbm, et_hbm, table_hbm, parts_hbm,
             etbuf, dstbuf, auxbuf, normbuf, rowsA, rowsB,
             ones_v, zline, gsemA, gsemB, ssemA, ssemB,
             deg_sh, acc_sh):
    s = lax.axis_index("s")

    zero16 = jnp.zeros((16,), jnp.float32)

    def _zrow(i, _):
        for k in range(8):
            rowsA[i, pl.ds(k * 16, 16)] = zero16
        return 0
    lax.fori_loop(0, _CH, _zrow, 0)
    for k in range(8):
        ones_v[pl.ds(k * 16, 16)] = jnp.full((16,), 1.0, jnp.float32)
    for k in range(64):
        zline[pl.ds(k * 16, 16)] = zero16

    dslice = _DEGSZ // _NS
    for k in range(5):
        pltpu.sync_copy(zline, deg_sh.at[pl.ds(s * dslice + k * 1024, 1024)])
    pltpu.sync_copy(zline.at[pl.ds(0, 512)],
                    deg_sh.at[pl.ds(s * dslice + 5 * 1024, 512)])
    aslice = _ACCROWS // _NS
    for k in range(4):
        pltpu.sync_copy(rowsA, acc_sh.at[pl.ds(s * aslice + k * _CH, _CH)])
    pltpu.sync_copy(rowsA.at[pl.ds(0, aslice - 4 * _CH)],
                    acc_sh.at[pl.ds(s * aslice + 4 * _CH, aslice - 4 * _CH)])
    plsc.subcore_barrier()

    h0 = s * _CPS

    def _hist_blk(b, _):
        base = h0 + b * _BR
        pltpu.sync_copy(et_hbm.at[pl.ds(base, _BR)], etbuf)
        pltpu.sync_copy(dst_hbm.at[pl.ds(base, _BR)], dstbuf)

        def _comb_row(j, _):
            for k in range(8):
                sl = pl.ds(k * 16, 16)
                auxbuf[j, sl] = etbuf[j, sl] * _N + dstbuf[j, sl]
            return 0
        lax.fori_loop(0, _BR, _comb_row, 0)

        def _hist_row(j, _):
            pltpu.sync_copy(ones_v, deg_sh.at[auxbuf.at[j]], add=True)
            return 0
        lax.fori_loop(0, _BR, _hist_row, 0)
        return 0
    lax.fori_loop(0, _NBLK, _hist_blk, 0)
    plsc.subcore_barrier()

    def _main_blk(b, _):
        base = h0 + b * _BR
        pltpu.sync_copy(et_hbm.at[pl.ds(base, _BR)], etbuf)
        pltpu.sync_copy(dst_hbm.at[pl.ds(base, _BR)], dstbuf)

        def _comb_row(j, _):
            for k in range(8):
                sl = pl.ds(k * 16, 16)
                auxbuf[j, sl] = etbuf[j, sl] * _N + dstbuf[j, sl]
            return 0
        lax.fori_loop(0, _BR, _comb_row, 0)

        def _deg_row(j, _):
            pltpu.sync_copy(deg_sh.at[auxbuf.at[j]], normbuf.at[j])
            return 0
        lax.fori_loop(0, _BR, _deg_row, 0)

        def _norm_row(j, _):
            for k in range(8):
                sl = pl.ds(k * 16, 16)
                normbuf[j, sl] = 1.0 / normbuf[j, sl]
            return 0
        lax.fori_loop(0, _BR, _norm_row, 0)

        pltpu.sync_copy(src_hbm.at[pl.ds(base, _BR)], auxbuf)

        def _idx_row(j, _):
            for k in range(8):
                sl = pl.ds(k * 16, 16)
                etbuf[j, sl] = etbuf[j, sl] * _N + auxbuf[j, sl]
            return 0
        lax.fori_loop(0, _BR, _idx_row, 0)

        def _scale(buf, j):
            def _sc16(eb, _):
                nv = normbuf[j, pl.ds(eb * 16, 16)]
                for i in range(16):
                    e = eb * 16 + i
                    n = nv[i]
                    for k in range(8):
                        sl = pl.ds(k * 16, 16)
                        buf[e, sl] = buf[e, sl] * n
                return 0
            lax.fori_loop(0, _CH // 16, _sc16, 0)

        def _gs(j, buf, sem):
            pltpu.async_copy(table_hbm.at[etbuf.at[j]], buf, sem)

        def _gw(j, buf, sem):
            pltpu.make_async_copy(table_hbm.at[etbuf.at[j]], buf, sem).wait()

        def _ss(j, buf, sem):
            pltpu.async_copy(buf, acc_sh.at[dstbuf.at[j]], sem, add=True)

        def _sw(j, buf, sem):
            pltpu.make_async_copy(buf, acc_sh.at[dstbuf.at[j]], sem).wait()

        _gs(0, rowsA, gsemA)
        _gs(1, rowsB, gsemB)

        def _pair(p, _):
            j0 = 2 * p
            _gw(j0, rowsA, gsemA)
            _scale(rowsA, j0)
            _ss(j0, rowsA, ssemA)
            _gw(j0 + 1, rowsB, gsemB)
            _scale(rowsB, j0 + 1)
            _ss(j0 + 1, rowsB, ssemB)
            _sw(j0, rowsA, ssemA)
            _gs(j0 + 2, rowsA, gsemA)
            _sw(j0 + 1, rowsB, ssemB)
            _gs(j0 + 3, rowsB, gsemB)
            return 0
        lax.fori_loop(0, _BR // 2 - 1, _pair, 0)

        jl = _BR - 2
        _gw(jl, rowsA, gsemA)
        _scale(rowsA, jl)
        _ss(jl, rowsA, ssemA)
        _gw(jl + 1, rowsB, gsemB)
        _scale(rowsB, jl + 1)
        _ss(jl + 1, rowsB, ssemB)
        _sw(jl, rowsA, ssemA)
        _sw(jl + 1, rowsB, ssemB)
        return 0
    lax.fori_loop(0, _NBLK, _main_blk, 0)
    plsc.subcore_barrier()

    pltpu.sync_copy(acc_sh.at[pl.ds(s * aslice, aslice)],
                    parts_hbm.at[pl.ds(s * aslice, aslice)])


_sc_kernel = functools.partial(
    pl.kernel,
    out_type=jax.ShapeDtypeStruct((_ACCROWS, _D), jnp.float32),
    mesh=plsc.VectorSubcoreMesh(core_axis_name="c", subcore_axis_name="s",
                                num_cores=_NC, num_subcores=_NS),
    scratch_types=[
        pltpu.VMEM((_BR, _CH), jnp.int32),
        pltpu.VMEM((_BR, _CH), jnp.int32),
        pltpu.VMEM((_BR, _CH), jnp.int32),
        pltpu.VMEM((_BR, _CH), jnp.float32),
        pltpu.VMEM((_CH, _D), jnp.float32),
        pltpu.VMEM((_CH, _D), jnp.float32),
        pltpu.VMEM((_CH,), jnp.float32),
        pltpu.VMEM((1024,), jnp.float32),
        pltpu.SemaphoreType.DMA,
        pltpu.SemaphoreType.DMA,
        pltpu.SemaphoreType.DMA,
        pltpu.SemaphoreType.DMA,
        pltpu.VMEM_SHARED((_DEGSZ,), jnp.float32),
        pltpu.VMEM_SHARED((_ACCROWS, _D), jnp.float32),
    ],
)(_sc_body)


def kernel(x, edge_index, edge_type, W, root, bias):
    src = edge_index[0]
    dst = edge_index[1]
    npad = _EPAD - _E
    srcp = jnp.pad(src, (0, npad)).reshape(_ROWS2D, _CH)
    dstp = jnp.pad(dst, (0, npad), constant_values=_N).reshape(_ROWS2D, _CH)
    etp = jnp.pad(edge_type, (0, npad),
                  constant_values=_R).reshape(_ROWS2D, _CH)
    wall = jnp.concatenate([W, root[None]], axis=0)

    table = _transform(x, wall)
    parts = _sc_kernel(srcp, dstp, etp, table)
    return _finalize(parts, table, bias.reshape(1, _D))


# speedup vs baseline: 10.0978x; 1.0161x over previous
import functools

import jax
import jax.numpy as jnp
from jax import lax
from jax.experimental import pallas as pl
from jax.experimental.pallas import tpu as pltpu
from jax.experimental.pallas import tpu_sc as plsc

_N = 10000
_E = 320000
_D = 128
_R = 8

_NC = 1
_NS = 16

_CH = 128
_CPS = 160
_EPAD = _NC * _NS * _CPS * _CH
_ROWS2D = _EPAD // _CH

_DEGSZ = 90112
_ACCROWS = 10112
_TBLROWS = (_R + 1) * _N


def _mm_body(x_ref, w_ref, o_ref):
    o_ref[...] = jnp.dot(x_ref[...], w_ref[0],
                         preferred_element_type=jnp.float32)


def _transform(x, wall):
    nblk = _N // 1000
    return pl.pallas_call(
        _mm_body,
        grid=(_R + 1, nblk),
        in_specs=[
            pl.BlockSpec((1000, _D), lambda r, i: (i, 0)),
            pl.BlockSpec((1, _D, _D), lambda r, i: (r, 0, 0)),
        ],
        out_specs=pl.BlockSpec((1000, _D), lambda r, i: (r * nblk + i, 0)),
        out_shape=jax.ShapeDtypeStruct((_TBLROWS, _D), jnp.float32),
    )(x, wall)


def _final_body(p_ref, t_ref, b_ref, o_ref):
    o_ref[...] = p_ref[...] + t_ref[...] + b_ref[...]


def _finalize(parts, table, bias):
    nblk = _N // 1000
    return pl.pallas_call(
        _final_body,
        grid=(nblk,),
        in_specs=[
            pl.BlockSpec((1000, _D), lambda i: (i, 0)),
            pl.BlockSpec((1000, _D), lambda i: (_R * nblk + i, 0)),
            pl.BlockSpec((1, _D), lambda i: (0, 0)),
        ],
        out_specs=pl.BlockSpec((1000, _D), lambda i: (i, 0)),
        out_shape=jax.ShapeDtypeStruct((_N, _D), jnp.float32),
    )(parts, table, bias)


_BR = 16
_NBLK = _CPS // _BR


def _sc_body(src_hbm, dst_hbm, et_hbm, table_hbm, parts_hbm,
             etbuf, dstbuf, auxbuf, normbuf, rowsA, rowsB,
             ones_v, zline, gsemA, gsemB, ssemA, ssemB, bsem,
             deg_sh, acc_sh):
    s = lax.axis_index("s")

    zero16 = jnp.zeros((16,), jnp.float32)

    def _zrow(i, _):
        for k in range(8):
            rowsA[i, pl.ds(k * 16, 16)] = zero16
        return 0
    lax.fori_loop(0, _CH, _zrow, 0)

    for k in range(8):
        ones_v[pl.ds(k * 16, 16)] = jnp.full((16,), 1.0, jnp.float32)
    for k in range(64):
        zline[pl.ds(k * 16, 16)] = zero16

    dslice = _DEGSZ // _NS
    for k in range(5):
        pltpu.sync_copy(zline, deg_sh.at[pl.ds(s * dslice + k * 1024, 1024)])
    pltpu.sync_copy(zline.at[pl.ds(0, 512)],
                    deg_sh.at[pl.ds(s * dslice + 5 * 1024, 512)])
    aslice = _ACCROWS // _NS
    for k in range(4):
        pltpu.sync_copy(rowsA, acc_sh.at[pl.ds(s * aslice + k * _CH, _CH)])
    pltpu.sync_copy(rowsA.at[pl.ds(0, aslice - 4 * _CH)],
                    acc_sh.at[pl.ds(s * aslice + 4 * _CH, aslice - 4 * _CH)])
    plsc.subcore_barrier()

    h0 = s * _CPS

    def _hist_blk(b, _):
        base = h0 + b * _BR
        pltpu.sync_copy(et_hbm.at[pl.ds(base, _BR)], etbuf)
        pltpu.sync_copy(dst_hbm.at[pl.ds(base, _BR)], dstbuf)

        def _comb_row(j, _):
            for k in range(8):
                sl = pl.ds(k * 16, 16)
                auxbuf[j, sl] = etbuf[j, sl] * _N + dstbuf[j, sl]
            return 0
        lax.fori_loop(0, _BR, _comb_row, 0)

        def _hfire(j, _):
            pltpu.async_copy(ones_v, deg_sh.at[auxbuf.at[j]], bsem, add=True)
            return 0
        lax.fori_loop(0, _BR, _hfire, 0)

        def _hdrain(j, _):
            pltpu.make_async_copy(ones_v, deg_sh.at[auxbuf.at[0]],
                                  bsem).wait()
            return 0
        lax.fori_loop(0, _BR, _hdrain, 0)
        return 0
    lax.fori_loop(0, _NBLK, _hist_blk, 0)
    plsc.subcore_barrier()

    def _main_blk(b, _):
        base = h0 + b * _BR
        pltpu.sync_copy(et_hbm.at[pl.ds(base, _BR)], etbuf)
        pltpu.sync_copy(dst_hbm.at[pl.ds(base, _BR)], dstbuf)

        def _comb_row(j, _):
            for k in range(8):
                sl = pl.ds(k * 16, 16)
                auxbuf[j, sl] = etbuf[j, sl] * _N + dstbuf[j, sl]
            return 0
        lax.fori_loop(0, _BR, _comb_row, 0)

        def _dfire(j, _):
            pltpu.async_copy(deg_sh.at[auxbuf.at[j]], normbuf.at[j], bsem)
            return 0
        lax.fori_loop(0, _BR, _dfire, 0)

        def _ddrain(j, _):
            pltpu.make_async_copy(deg_sh.at[auxbuf.at[0]], normbuf.at[0],
                                  bsem).wait()
            return 0
        lax.fori_loop(0, _BR, _ddrain, 0)

        def _norm_row(j, _):
            for k in range(8):
                sl = pl.ds(k * 16, 16)
                normbuf[j, sl] = 1.0 / normbuf[j, sl]
            return 0
        lax.fori_loop(0, _BR, _norm_row, 0)

        pltpu.sync_copy(src_hbm.at[pl.ds(base, _BR)], auxbuf)

        def _idx_row(j, _):
            for k in range(8):
                sl = pl.ds(k * 16, 16)
                etbuf[j, sl] = etbuf[j, sl] * _N + auxbuf[j, sl]
            return 0
        lax.fori_loop(0, _BR, _idx_row, 0)

        def _scale(buf, j):
            def _sc16(eb, _):
                nv = normbuf[j, pl.ds(eb * 16, 16)]
                for i in range(16):
                    e = eb * 16 + i
                    n = nv[i]
                    for k in range(8):
                        sl = pl.ds(k * 16, 16)
                        buf[e, sl] = buf[e, sl] * n
                return 0
            lax.fori_loop(0, _CH // 16, _sc16, 0)

        def _gs(j, buf, sem):
            pltpu.async_copy(table_hbm.at[etbuf.at[j]], buf, sem)

        def _gw(j, buf, sem):
            pltpu.make_async_copy(table_hbm.at[etbuf.at[j]], buf, sem).wait()

        def _ss(j, buf, sem):
            pltpu.async_copy(buf, acc_sh.at[dstbuf.at[j]], sem, add=True)

        def _sw(j, buf, sem):
            pltpu.make_async_copy(buf, acc_sh.at[dstbuf.at[j]], sem).wait()

        _gs(0, rowsA, gsemA)
        _gs(1, rowsB, gsemB)

        def _pair(p, _):
            j0 = 2 * p
            _gw(j0, rowsA, gsemA)
            _scale(rowsA, j0)
            _ss(j0, rowsA, ssemA)
            _gw(j0 + 1, rowsB, gsemB)
            _scale(rowsB, j0 + 1)
            _ss(j0 + 1, rowsB, ssemB)
            _sw(j0, rowsA, ssemA)
            _gs(j0 + 2, rowsA, gsemA)
            _sw(j0 + 1, rowsB, ssemB)
            _gs(j0 + 3, rowsB, gsemB)
            return 0
        lax.fori_loop(0, _BR // 2 - 1, _pair, 0)

        jl = _BR - 2
        _gw(jl, rowsA, gsemA)
        _scale(rowsA, jl)
        _ss(jl, rowsA, ssemA)
        _gw(jl + 1, rowsB, gsemB)
        _scale(rowsB, jl + 1)
        _ss(jl + 1, rowsB, ssemB)
        _sw(jl, rowsA, ssemA)
        _sw(jl + 1, rowsB, ssemB)
        return 0
    lax.fori_loop(0, _NBLK, _main_blk, 0)
    plsc.subcore_barrier()

    pltpu.sync_copy(acc_sh.at[pl.ds(s * aslice, aslice)],
                    parts_hbm.at[pl.ds(s * aslice, aslice)])


_sc_kernel = functools.partial(
    pl.kernel,
    out_type=jax.ShapeDtypeStruct((_ACCROWS, _D), jnp.float32),
    mesh=plsc.VectorSubcoreMesh(core_axis_name="c", subcore_axis_name="s",
                                num_cores=_NC, num_subcores=_NS),
    scratch_types=[
        pltpu.VMEM((_BR, _CH), jnp.int32),
        pltpu.VMEM((_BR, _CH), jnp.int32),
        pltpu.VMEM((_BR, _CH), jnp.int32),
        pltpu.VMEM((_BR, _CH), jnp.float32),
        pltpu.VMEM((_CH, _D), jnp.float32),
        pltpu.VMEM((_CH, _D), jnp.float32),
        pltpu.VMEM((_CH,), jnp.float32),
        pltpu.VMEM((1024,), jnp.float32),
        pltpu.SemaphoreType.DMA,
        pltpu.SemaphoreType.DMA,
        pltpu.SemaphoreType.DMA,
        pltpu.SemaphoreType.DMA,
        pltpu.SemaphoreType.DMA,
        pltpu.VMEM_SHARED((_DEGSZ,), jnp.float32),
        pltpu.VMEM_SHARED((_ACCROWS, _D), jnp.float32),
    ],
)(_sc_body)


def kernel(x, edge_index, edge_type, W, root, bias):
    src = edge_index[0]
    dst = edge_index[1]
    npad = _EPAD - _E
    srcp = jnp.pad(src, (0, npad)).reshape(_ROWS2D, _CH)
    dstp = jnp.pad(dst, (0, npad), constant_values=_N).reshape(_ROWS2D, _CH)
    etp = jnp.pad(edge_type, (0, npad),
                  constant_values=_R).reshape(_ROWS2D, _CH)
    wall = jnp.concatenate([W, root[None]], axis=0)

    table = _transform(x, wall)
    parts = _sc_kernel(srcp, dstp, etp, table)
    return _finalize(parts, table, bias.reshape(1, _D))
